# Initial kernel scaffold; baseline (speedup 1.0000x reference)
import jax
import jax.numpy as jnp
from jax.experimental import pallas as pl


def kernel(type_indices, embedding_weight):
    raise NotImplementedError("write your pallas kernel here")


# speedup vs baseline: 2.9365x; 2.9365x over previous
import functools

import jax
import jax.numpy as jnp
from jax import lax
from jax.experimental import pallas as pl
from jax.experimental.pallas import tpu as pltpu
from jax.experimental.pallas import tpu_sc as plsc

N_TYPES = 3
EMB_D = 64
QUAD = 4
QD = EMB_D * QUAD
QCHUNK = 128


@functools.lru_cache(maxsize=None)
def _make_lookup(bq: int):
    info = plsc.get_sparse_core_info()
    nw = info.num_cores * info.num_subcores
    per_w = bq // nw
    n_chunks = per_w // QCHUNK
    assert bq % (nw * QCHUNK) == 0

    mesh = plsc.VectorSubcoreMesh(core_axis_name="c", subcore_axis_name="s")

    @functools.partial(
        pl.kernel,
        mesh=mesh,
        out_type=jax.ShapeDtypeStruct((bq, QD), jnp.float32),
        scratch_types=[
            pltpu.VMEM((QCHUNK,), jnp.int32),
            pltpu.VMEM((QCHUNK, QD), jnp.float32),
            pltpu.SemaphoreType.DMA,
        ],
    )
    def lookup(qtbl_hbm, qid_hbm, out_hbm, qid_v, rows_v, sem):
        wid = lax.axis_index("s") * info.num_cores + lax.axis_index("c")
        base0 = wid * per_w

        def chunk_body(i, carry):
            qbase = base0 + i * QCHUNK
            pltpu.sync_copy(qid_hbm.at[pl.ds(qbase, QCHUNK)], qid_v)
            pltpu.async_copy(qtbl_hbm.at[qid_v], rows_v, sem).wait()
            pltpu.sync_copy(rows_v, out_hbm.at[pl.ds(qbase, QCHUNK)])
            return carry

        lax.fori_loop(0, n_chunks, chunk_body, 0)

    return lookup


def kernel(type_indices, embedding_weight):
    b, s = type_indices.shape
    quads = type_indices.reshape(b * s // QUAD, QUAD).astype(jnp.int32)
    weights = jnp.array([N_TYPES**(QUAD - 1 - k) for k in range(QUAD)], jnp.int32)
    qid = quads @ weights
    q = jnp.arange(N_TYPES**QUAD)
    digits = jnp.stack(
        [(q // (N_TYPES**(QUAD - 1 - k))) % N_TYPES for k in range(QUAD)], axis=-1
    )
    qtbl = embedding_weight[digits].reshape(N_TYPES**QUAD, QD)
    out = _make_lookup(b * s // QUAD)(qtbl, qid)
    return out.reshape(b, s, EMB_D)
